# baseline (speedup 1.0000x reference)
import functools
import math

import jax
import jax.numpy as jnp
from jax import lax
from jax.experimental import pallas as pl
from jax.experimental.pallas import tpu as pltpu
from jax.experimental.pallas import tpu_sc as plsc

_B = 64
_S = 512
_N = _B * _S
_D4 = 32
_D = 128

_NC = 2
_NS = 16
_NW = _NC * _NS
_B_PER_W = _N // _NW
_CHUNK = 128
_NCHUNK = _B_PER_W // _CHUNK

_SPB = 8


def _sc_gather_kernel(table_hbm, idx_hbm, out_hbm, idx_v, buf_v, sem):
    wid = lax.axis_index("s") * _NC + lax.axis_index("c")
    base = wid * _B_PER_W
    pltpu.sync_copy(idx_hbm.at[pl.ds(base, _B_PER_W)], idx_v)
    copies = []
    for j in range(_NCHUNK):
        copies.append(
            pltpu.async_copy(
                table_hbm.at[idx_v.at[pl.ds(j * _CHUNK, _CHUNK)]],
                buf_v.at[j % 2],
                sem,
            )
        )
        if j > 0:
            copies[j - 1].wait()
            pltpu.sync_copy(
                buf_v.at[(j - 1) % 2],
                out_hbm.at[pl.ds(base + (j - 1) * _CHUNK, _CHUNK)],
            )
    copies[_NCHUNK - 1].wait()
    pltpu.sync_copy(
        buf_v.at[(_NCHUNK - 1) % 2],
        out_hbm.at[pl.ds(base + (_NCHUNK - 1) * _CHUNK, _CHUNK)],
    )


def _sc_gather(tablep, idx_flat):
    mesh = plsc.VectorSubcoreMesh(core_axis_name="c", subcore_axis_name="s")
    k = pl.kernel(
        _sc_gather_kernel,
        out_type=jax.ShapeDtypeStruct((_N, _D), jnp.float32),
        mesh=mesh,
        compiler_params=pltpu.CompilerParams(use_tc_tiling_on_sc=True),
        scratch_types=[
            pltpu.VMEM((_B_PER_W,), jnp.int32),
            pltpu.VMEM((2, _CHUNK, _D), jnp.float32),
            pltpu.SemaphoreType.DMA,
        ],
    )
    return k(tablep, idx_flat)


def _tc_body(g_ref, tt_ref, w_ref, b_ref, pos_ref, type_ref, gm_ref, bt_ref,
             out_ref):
    g = g_ref[...]
    lane = lax.broadcasted_iota(jnp.int32, (_SPB * _S, _D), 1)
    h = 0.5 * g * (1.0 + lax.erf(g * (1.0 / math.sqrt(2.0))))
    h = jnp.where(lane < _D4, h, 0.0)
    w = w_ref[...]
    wpad = jnp.concatenate([w, jnp.zeros((_D - _D4, _D), jnp.float32)], axis=0)
    h2 = jnp.dot(h, wpad, preferred_element_type=jnp.float32)
    h3 = h2.reshape(_SPB, _S, _D)
    h3 = h3 + b_ref[...].reshape(1, 1, _D)
    h3 = h3 + pos_ref[...][None, :, :]
    ty0 = type_ref[0, :].reshape(1, 1, _D)
    tyd = (type_ref[1, :] - type_ref[0, :]).reshape(1, 1, _D)
    tt = tt_ref[...].astype(jnp.float32)
    h3 = h3 + ty0 + tt[:, :, None] * tyd
    mean = jnp.mean(h3, axis=-1, keepdims=True)
    d = h3 - mean
    var = jnp.mean(d * d, axis=-1, keepdims=True)
    out_ref[...] = (d * lax.rsqrt(var + 1e-12)) * gm_ref[...].reshape(1, 1, _D) \
        + bt_ref[...].reshape(1, 1, _D)


def _tc_compute(gathered, token_type, proj_W, proj_b, pos, type_table, gamma,
                beta):
    grid = (_B // _SPB,)
    full = lambda i: (0, 0)
    return pl.pallas_call(
        _tc_body,
        grid=grid,
        in_specs=[
            pl.BlockSpec((_SPB * _S, _D), lambda i: (i, 0)),
            pl.BlockSpec((_SPB, _S), lambda i: (i, 0)),
            pl.BlockSpec((_D4, _D), full),
            pl.BlockSpec((1, _D), full),
            pl.BlockSpec((_S, _D), full),
            pl.BlockSpec((2, _D), full),
            pl.BlockSpec((1, _D), full),
            pl.BlockSpec((1, _D), full),
        ],
        out_specs=pl.BlockSpec((_SPB, _S, _D), lambda i: (i, 0, 0)),
        out_shape=jax.ShapeDtypeStruct((_B, _S, _D), jnp.float32),
    )(gathered, token_type, proj_W, proj_b, pos, type_table, gamma, beta)


def kernel(x, token_type, token_table, proj_W, proj_b, pos_table, type_table,
           gamma, beta):
    tablep = jnp.pad(token_table, ((0, 0), (0, _D - _D4)))
    idx_flat = x.reshape(_N)
    gathered = _sc_gather(tablep, idx_flat)
    return _tc_compute(
        gathered,
        token_type,
        proj_W,
        proj_b.reshape(1, _D),
        pos_table[:_S],
        type_table,
        gamma.reshape(1, _D),
        beta.reshape(1, _D),
    )
